# baseline (speedup 1.0000x reference)
import functools

import jax
import jax.numpy as jnp
from jax import lax
from jax.experimental import pallas as pl
from jax.experimental.pallas import tpu as pltpu
from jax.experimental.pallas import tpu_sc as plsc

N = 10000
NTILES = 16
ROWS_PER_TEC = 632
NPAD = NTILES * ROWS_PER_TEC
E = 160000
K = 128
NB = 80
EPT = 10240
EPAD = NTILES * EPT
C = 128
BN = 400

_MESH = plsc.VectorSubcoreMesh(
    core_axis_name="c", subcore_axis_name="s", num_cores=2, num_subcores=16)


def _fill(buf, w, value, dtype=jnp.float32):
    if dtype == jnp.bfloat16:
        @pl.loop(0, buf.shape[0] // 2)
        def _(i):
            r = pl.multiple_of(2 * i, 2)
            for j in range(w // 16):
                buf[pl.ds(r, 2), pl.ds(j * 16, 16)] = jnp.full(
                    (2, 16), value, dtype)
    else:
        @pl.loop(0, buf.shape[0])
        def _(i):
            for j in range(w // 16):
                buf[i, pl.ds(j * 16, 16)] = jnp.full((16,), value, dtype)


def _row_pieces(nrows):
    full, rem = divmod(ROWS_PER_TEC, nrows)
    pieces = [(k * nrows, nrows) for k in range(full)]
    if rem:
        pieces.append((full * nrows, rem))
    return pieces


def _zero_rows(zbuf, acc, r0):
    for off, n in _row_pieces(zbuf.shape[0]):
        pltpu.sync_copy(zbuf.at[pl.ds(0, n)], acc.at[pl.ds(r0 + off, n)])


def _copy_out(acc, bounce, out_hbm, r0):
    for off, n in _row_pieces(bounce.shape[0]):
        pltpu.sync_copy(acc.at[pl.ds(r0 + off, n)], bounce.at[pl.ds(0, n)])
        pltpu.sync_copy(bounce.at[pl.ds(0, n)], out_hbm.at[pl.ds(r0 + off, n)])


@functools.partial(
    pl.kernel,
    out_type=(jax.ShapeDtypeStruct((NPAD, 16), jnp.float32),
              jax.ShapeDtypeStruct((NPAD, 16), jnp.float32)),
    mesh=_MESH,
    scratch_types=[
        pltpu.VMEM((NB, K), jnp.int32),
        pltpu.VMEM((K, 16), jnp.float32),
        pltpu.VMEM((K, 16), jnp.float32),
        pltpu.VMEM_SHARED((NPAD, 16), jnp.float32),
    ],
)
def _deg_kernel(dstg, dstd, outg, outd, didx2, ones_b, z_b, acc):
    cid = lax.axis_index("c")
    sid = lax.axis_index("s")
    r0 = sid * ROWS_PER_TEC
    _fill(ones_b, 16, 1.0)
    _fill(z_b, 16, 0.0)
    _zero_rows(z_b, acc, r0)
    plsc.subcore_barrier()

    def scat(dst3):
        pltpu.sync_copy(dst3.at[sid], didx2)

        @pl.loop(0, NB)
        def _(b):
            pltpu.sync_copy(ones_b, acc.at[didx2.at[b]], add=True)

    @pl.when(cid == 0)
    def _():
        scat(dstg)

    @pl.when(cid == 1)
    def _():
        scat(dstd)

    plsc.subcore_barrier()

    @pl.when(cid == 0)
    def _():
        _copy_out(acc, ones_b, outg, r0)

    @pl.when(cid == 1)
    def _():
        _copy_out(acc, ones_b, outd, r0)


@functools.lru_cache(maxsize=None)
def _make_agg(nc):
    out_type = tuple(jax.ShapeDtypeStruct((NPAD, C), jnp.float32)
                     for _ in range(2 * nc))
    GD = 16
    NG = NB // GD
    scratch = [
        pltpu.VMEM((EPT,), jnp.int32),
        pltpu.VMEM((GD, K), jnp.int32),
        pltpu.VMEM((GD, K), jnp.int32),
        pltpu.VMEM((2, K, C), jnp.float32),
        pltpu.VMEM_SHARED((NPAD, C), jnp.float32),
    ] + [pltpu.SemaphoreType.DMA] * 4

    def body(*refs):
        hps = refs[:2 * nc]
        srcg, dstg, srcd, dstd = refs[2 * nc:2 * nc + 4]
        outs = refs[2 * nc + 4:4 * nc + 4]
        sidx = refs[4 * nc + 4]
        didx = refs[4 * nc + 5:4 * nc + 7]
        ring, acc = refs[4 * nc + 7:4 * nc + 9]
        isems = refs[4 * nc + 9:4 * nc + 11]
        gsems = refs[4 * nc + 11:4 * nc + 13]
        cid = lax.axis_index("c")
        sid = lax.axis_index("s")
        r0 = sid * ROWS_PER_TEC

        def dstart(dh, g, sl):
            pltpu.async_copy(dh.at[sid, pl.ds(g * GD, GD)], didx[sl],
                             isems[sl])

        def dwait(dh, g, sl):
            pltpu.make_async_copy(dh.at[sid, pl.ds(g * GD, GD)], didx[sl],
                                  isems[sl]).wait()

        def gstart(hp, b, sl):
            pltpu.async_copy(hp.at[sidx.at[pl.ds(b * K, K)]], ring.at[sl],
                             gsems[sl])

        def gwait(hp, b, sl):
            pltpu.make_async_copy(hp.at[sidx.at[pl.ds(b * K, K)]],
                                  ring.at[sl], gsems[sl]).wait()

        def run_pass(hp, sh, dh):
            pltpu.sync_copy(sh.at[sid], sidx)
            dstart(dh, 0, 0)
            for g in range(NG):
                sl = g % 2
                dwait(dh, g, sl)
                if g + 1 < NG:
                    dstart(dh, g + 1, 1 - sl)
                b_base = g * GD
                gstart(hp, b_base, 0)

                @pl.loop(0, GD // 2)
                def _(i, sl=sl, b_base=b_base):
                    b0 = b_base + 2 * i
                    gstart(hp, b0 + 1, 1)
                    gwait(hp, b0, 0)
                    pltpu.sync_copy(ring.at[0], acc.at[didx[sl].at[2 * i]],
                                    add=True)

                    @pl.when(i < GD // 2 - 1)
                    def _():
                        gstart(hp, b0 + 2, 0)

                    gwait(hp, b0 + 1, 1)
                    pltpu.sync_copy(ring.at[1], acc.at[didx[sl].at[2 * i + 1]],
                                    add=True)

        chunks = [(g, c) for g in (0, 1) for c in range(nc)]
        sched = (chunks[0::2], chunks[1::2])
        idx_pairs = ((srcg, dstg), (srcd, dstd))
        zb = ring.at[0]
        for p in range(nc):
            _fill(zb, C, 0.0)
            for off, n in _row_pieces(K):
                pltpu.async_copy(zb.at[pl.ds(0, n)],
                                 acc.at[pl.ds(r0 + off, n)], gsems[0])
            for off, n in _row_pieces(K):
                pltpu.make_async_copy(zb.at[pl.ds(0, n)],
                                      acc.at[pl.ds(r0 + off, n)],
                                      gsems[0]).wait()
            plsc.subcore_barrier()
            for core in (0, 1):
                g, c = sched[core][p]

                @pl.when(cid == core)
                def _(hp=hps[g * nc + c], sh=idx_pairs[g][0],
                      dh=idx_pairs[g][1]):
                    run_pass(hp, sh, dh)

            plsc.subcore_barrier()
            for core in (0, 1):
                g, c = sched[core][p]

                @pl.when(cid == core)
                def _(out=outs[g * nc + c]):
                    pieces = _row_pieces(K)
                    for k, (off, n) in enumerate(pieces):
                        sl = k % 2
                        if k >= 2:
                            poff, pn = pieces[k - 2]
                            pltpu.make_async_copy(
                                ring.at[sl].at[pl.ds(0, pn)],
                                out.at[pl.ds(r0 + poff, pn)],
                                isems[sl]).wait()
                        pltpu.async_copy(acc.at[pl.ds(r0 + off, n)],
                                         ring.at[sl].at[pl.ds(0, n)],
                                         gsems[sl])
                        pltpu.make_async_copy(acc.at[pl.ds(r0 + off, n)],
                                              ring.at[sl].at[pl.ds(0, n)],
                                              gsems[sl]).wait()
                        pltpu.async_copy(ring.at[sl].at[pl.ds(0, n)],
                                         out.at[pl.ds(r0 + off, n)],
                                         isems[sl])
                    for k in (len(pieces) - 2, len(pieces) - 1):
                        off, n = pieces[k]
                        pltpu.make_async_copy(ring.at[k % 2].at[pl.ds(0, n)],
                                              out.at[pl.ds(r0 + off, n)],
                                              isems[k % 2]).wait()

    return pl.kernel(body, out_type=out_type, mesh=_MESH,
                     scratch_types=scratch)


def _tcpre(r, Wg, Wd, cntg, cntd, apply_relu, cw):
    din = r.shape[1]
    dout = Wg.shape[1]
    nc = dout // cw

    def body(r_ref, wg_ref, wd_ref, cg_ref, cd_ref, *out_refs):
        rb = r_ref[...]
        if apply_relu:
            rb = jnp.maximum(rb, 0.0)
        disg = lax.rsqrt(1.0 + cg_ref[...])
        disd = lax.rsqrt(1.0 + cd_ref[...])
        hg = jnp.dot(rb, wg_ref[...], preferred_element_type=jnp.float32) * disg
        hd = jnp.dot(rb, wd_ref[...], preferred_element_type=jnp.float32) * disd
        for c in range(nc):
            out_refs[c][...] = hg[:, c * cw:(c + 1) * cw]
            out_refs[nc + c][...] = hd[:, c * cw:(c + 1) * cw]

    return pl.pallas_call(
        body,
        grid=(N // BN,),
        in_specs=[
            pl.BlockSpec((BN, din), lambda i: (i, 0)),
            pl.BlockSpec((din, dout), lambda i: (0, 0)),
            pl.BlockSpec((din, dout), lambda i: (0, 0)),
            pl.BlockSpec((BN, 1), lambda i: (i, 0)),
            pl.BlockSpec((BN, 1), lambda i: (i, 0)),
        ],
        out_specs=[pl.BlockSpec((BN, cw), lambda i: (i, 0))] * (2 * nc),
        out_shape=[jax.ShapeDtypeStruct((N, cw), jnp.float32)] * (2 * nc),
    )(r, Wg, Wd, cntg, cntd)


def _tc_mid(accs, hps, cntg, cntd, bg, bd, Wg2, Wd2, cw):
    nc = len(accs) // 2
    dout = nc * cw
    dout2 = Wg2.shape[1]
    nc2 = dout2 // cw

    def body(*refs):
        accr = refs[:2 * nc]
        hpr = refs[2 * nc:4 * nc]
        cg, cd, bgr, bdr, wg2, wd2 = refs[4 * nc:4 * nc + 6]
        xl = refs[4 * nc + 6]
        out2 = refs[4 * nc + 7:]
        disg = lax.rsqrt(1.0 + cg[...])
        disd = lax.rsqrt(1.0 + cd[...])
        bias = bgr[...] + bdr[...]
        cols = []
        for c in range(nc):
            cols.append(disg * (accr[c][...] + hpr[c][...])
                        + disd * (accr[nc + c][...] + hpr[nc + c][...])
                        + bias[:, c * cw:(c + 1) * cw])
        x = cols[0] if nc == 1 else jnp.concatenate(cols, axis=1)
        xl[...] = x
        rb = jnp.maximum(x, 0.0)
        hg = jnp.dot(rb, wg2[...], preferred_element_type=jnp.float32) * disg
        hd = jnp.dot(rb, wd2[...], preferred_element_type=jnp.float32) * disd
        for c in range(nc2):
            out2[c][...] = hg[:, c * cw:(c + 1) * cw]
            out2[nc2 + c][...] = hd[:, c * cw:(c + 1) * cw]

    in_specs = (
        [pl.BlockSpec((BN, cw), lambda i: (i, 0))] * (4 * nc)
        + [pl.BlockSpec((BN, 1), lambda i: (i, 0)),
           pl.BlockSpec((BN, 1), lambda i: (i, 0)),
           pl.BlockSpec((1, dout), lambda i: (0, 0)),
           pl.BlockSpec((1, dout), lambda i: (0, 0)),
           pl.BlockSpec((dout, dout2), lambda i: (0, 0)),
           pl.BlockSpec((dout, dout2), lambda i: (0, 0))]
    )
    out_specs = ([pl.BlockSpec((BN, dout), lambda i: (i, 0))]
                 + [pl.BlockSpec((BN, cw), lambda i: (i, 0))] * (2 * nc2))
    out_shape = ([jax.ShapeDtypeStruct((N, dout), jnp.float32)]
                 + [jax.ShapeDtypeStruct((N, cw), jnp.float32)] * (2 * nc2))
    res = pl.pallas_call(
        body,
        grid=(N // BN,),
        in_specs=in_specs,
        out_specs=out_specs,
        out_shape=out_shape,
    )(*accs, *hps, cntg, cntd, bg, bd, Wg2, Wd2)
    return res[0], tuple(res[1:])


def _tcpost(accs, hps, cntg, cntd, bg, bd, cw):
    nc = len(accs) // 2
    dout = nc * cw

    def body(*refs):
        accr = refs[:2 * nc]
        hpr = refs[2 * nc:4 * nc]
        cg, cd, bgr, bdr, out = refs[4 * nc:]
        disg = lax.rsqrt(1.0 + cg[...])
        disd = lax.rsqrt(1.0 + cd[...])
        bias = bgr[...] + bdr[...]
        cols = []
        for c in range(nc):
            xc = (disg * (accr[c][...] + hpr[c][...])
                  + disd * (accr[nc + c][...] + hpr[nc + c][...])
                  + bias[:, c * cw:(c + 1) * cw])
            cols.append(xc)
        out[...] = cols[0] if nc == 1 else jnp.concatenate(cols, axis=1)

    in_specs = (
        [pl.BlockSpec((BN, cw), lambda i: (i, 0))] * (2 * nc)
        + [pl.BlockSpec((BN, cw), lambda i: (i, 0))] * (2 * nc)
        + [pl.BlockSpec((BN, 1), lambda i: (i, 0)),
           pl.BlockSpec((BN, 1), lambda i: (i, 0)),
           pl.BlockSpec((1, dout), lambda i: (0, 0)),
           pl.BlockSpec((1, dout), lambda i: (0, 0))]
    )
    return pl.pallas_call(
        body,
        grid=(N // BN,),
        in_specs=in_specs,
        out_specs=pl.BlockSpec((BN, dout), lambda i: (i, 0)),
        out_shape=jax.ShapeDtypeStruct((N, dout), jnp.float32),
    )(*accs, *hps, cntg, cntd, bg, bd)


def _pad_edges(adj):
    src = adj[0].astype(jnp.int32)
    dst = adj[1].astype(jnp.int32)
    pad = EPAD - src.shape[0]
    src = jnp.concatenate([src, jnp.zeros((pad,), jnp.int32)])
    dst = jnp.concatenate([dst, jnp.full((pad,), N, jnp.int32)])
    return src.reshape(NTILES, EPT), dst.reshape(NTILES, NB, K)


def kernel(x, adj_t_gg, adj_t_gd, emb,
           W1_gg, b1_gg, W2_gg, b2_gg, W3_gg, b3_gg,
           W1_gd, b1_gd, W2_gd, b2_gd, W3_gd, b3_gd):
    xe = jnp.concatenate([x, emb], axis=0)
    srcg, dstg = _pad_edges(adj_t_gg)
    srcd, dstd = _pad_edges(adj_t_gd)

    cntg16, cntd16 = _deg_kernel(dstg, dstd)
    cntg = lax.slice(cntg16, (0, 0), (N, 1))
    cntd = lax.slice(cntd16, (0, 0), (N, 1))

    def agg(hps, nc):
        return tuple(_make_agg(nc)(*hps, srcg, dstg, srcd, dstd))

    hps1 = tuple(_tcpre(xe, W1_gg, W1_gd, cntg, cntd, apply_relu=False, cw=C))
    accs1 = agg(hps1, W1_gg.shape[1] // C)
    x1, hps2 = _tc_mid(accs1, hps1, cntg, cntd, b1_gg.reshape(1, -1),
                       b1_gd.reshape(1, -1), W2_gg, W2_gd, cw=C)
    accs2 = agg(hps2, W2_gg.shape[1] // C)
    x2, hps3 = _tc_mid(accs2, hps2, cntg, cntd, b2_gg.reshape(1, -1),
                       b2_gd.reshape(1, -1), W3_gg, W3_gd, cw=C)
    accs3 = agg(hps3, W3_gg.shape[1] // C)
    x3 = _tcpost(accs3, hps3, cntg, cntd, b3_gg.reshape(1, -1),
                 b3_gd.reshape(1, -1), cw=C)
    return jnp.concatenate([x1, x2, x3], axis=-1)


# speedup vs baseline: 1.0503x; 1.0503x over previous
import functools

import jax
import jax.numpy as jnp
from jax import lax
from jax.experimental import pallas as pl
from jax.experimental.pallas import tpu as pltpu
from jax.experimental.pallas import tpu_sc as plsc

N = 10000
NTILES = 16
ROWS_PER_TEC = 632
NPAD = NTILES * ROWS_PER_TEC
E = 160000
K = 128
NB = 80
EPT = 10240
EPAD = NTILES * EPT
C = 128
BN = 400

_MESH = plsc.VectorSubcoreMesh(
    core_axis_name="c", subcore_axis_name="s", num_cores=2, num_subcores=16)


def _fill(buf, w, value, dtype=jnp.float32):
    if dtype == jnp.bfloat16:
        @pl.loop(0, buf.shape[0] // 2)
        def _(i):
            r = pl.multiple_of(2 * i, 2)
            for j in range(w // 16):
                buf[pl.ds(r, 2), pl.ds(j * 16, 16)] = jnp.full(
                    (2, 16), value, dtype)
    else:
        @pl.loop(0, buf.shape[0])
        def _(i):
            for j in range(w // 16):
                buf[i, pl.ds(j * 16, 16)] = jnp.full((16,), value, dtype)


def _row_pieces(nrows):
    full, rem = divmod(ROWS_PER_TEC, nrows)
    pieces = [(k * nrows, nrows) for k in range(full)]
    if rem:
        pieces.append((full * nrows, rem))
    return pieces


def _zero_rows(zbuf, acc, r0):
    for off, n in _row_pieces(zbuf.shape[0]):
        pltpu.sync_copy(zbuf.at[pl.ds(0, n)], acc.at[pl.ds(r0 + off, n)])


def _copy_out(acc, bounce, out_hbm, r0):
    for off, n in _row_pieces(bounce.shape[0]):
        pltpu.sync_copy(acc.at[pl.ds(r0 + off, n)], bounce.at[pl.ds(0, n)])
        pltpu.sync_copy(bounce.at[pl.ds(0, n)], out_hbm.at[pl.ds(r0 + off, n)])


@functools.partial(
    pl.kernel,
    out_type=(jax.ShapeDtypeStruct((NPAD, 16), jnp.float32),
              jax.ShapeDtypeStruct((NPAD, 16), jnp.float32)),
    mesh=_MESH,
    scratch_types=[
        pltpu.VMEM((NB, K), jnp.int32),
        pltpu.VMEM((K, 16), jnp.float32),
        pltpu.VMEM((K, 16), jnp.float32),
        pltpu.VMEM_SHARED((NPAD, 16), jnp.float32),
    ],
)
def _deg_kernel(dstg, dstd, outg, outd, didx2, ones_b, z_b, acc):
    cid = lax.axis_index("c")
    sid = lax.axis_index("s")
    r0 = sid * ROWS_PER_TEC
    _fill(ones_b, 16, 1.0)
    _fill(z_b, 16, 0.0)
    _zero_rows(z_b, acc, r0)
    plsc.subcore_barrier()

    def scat(dst3):
        pltpu.sync_copy(dst3.at[sid], didx2)

        @pl.loop(0, NB)
        def _(b):
            pltpu.sync_copy(ones_b, acc.at[didx2.at[b]], add=True)

    @pl.when(cid == 0)
    def _():
        scat(dstg)

    @pl.when(cid == 1)
    def _():
        scat(dstd)

    plsc.subcore_barrier()

    @pl.when(cid == 0)
    def _():
        _copy_out(acc, ones_b, outg, r0)

    @pl.when(cid == 1)
    def _():
        _copy_out(acc, ones_b, outd, r0)


@functools.lru_cache(maxsize=None)
def _make_agg(nc):
    out_type = tuple(jax.ShapeDtypeStruct((NPAD, C), jnp.float32)
                     for _ in range(2 * nc))
    GD = 16
    NG = NB // GD
    scratch = [
        pltpu.VMEM((EPT,), jnp.int32),
        pltpu.VMEM((GD, K), jnp.int32),
        pltpu.VMEM((GD, K), jnp.int32),
        pltpu.VMEM((2, K, C), jnp.float32),
        pltpu.VMEM_SHARED((NPAD, C), jnp.float32),
    ] + [pltpu.SemaphoreType.DMA] * 4

    def body(*refs):
        hps = refs[:2 * nc]
        srcg, dstg, srcd, dstd = refs[2 * nc:2 * nc + 4]
        outs = refs[2 * nc + 4:4 * nc + 4]
        sidx = refs[4 * nc + 4]
        didx = refs[4 * nc + 5:4 * nc + 7]
        ring, acc = refs[4 * nc + 7:4 * nc + 9]
        isems = refs[4 * nc + 9:4 * nc + 11]
        gsems = refs[4 * nc + 11:4 * nc + 13]
        cid = lax.axis_index("c")
        sid = lax.axis_index("s")
        r0 = sid * ROWS_PER_TEC

        def dstart(dh, g, sl):
            pltpu.async_copy(dh.at[sid, pl.ds(g * GD, GD)], didx[sl],
                             isems[sl])

        def dwait(dh, g, sl):
            pltpu.make_async_copy(dh.at[sid, pl.ds(g * GD, GD)], didx[sl],
                                  isems[sl]).wait()

        def gstart(hp, b, sl):
            pltpu.async_copy(hp.at[sidx.at[pl.ds(b * K, K)]], ring.at[sl],
                             gsems[sl])

        def gwait(hp, b, sl):
            pltpu.make_async_copy(hp.at[sidx.at[pl.ds(b * K, K)]],
                                  ring.at[sl], gsems[sl]).wait()

        def run_pass(hp, sh, dh):
            pltpu.sync_copy(sh.at[sid], sidx)
            dstart(dh, 0, 0)
            for g in range(NG):
                sl = g % 2
                dwait(dh, g, sl)
                if g + 1 < NG:
                    dstart(dh, g + 1, 1 - sl)
                b_base = g * GD
                gstart(hp, b_base, 0)

                @pl.loop(0, GD // 2)
                def _(i, sl=sl, b_base=b_base):
                    b0 = b_base + 2 * i
                    gstart(hp, b0 + 1, 1)
                    gwait(hp, b0, 0)
                    pltpu.sync_copy(ring.at[0], acc.at[didx[sl].at[2 * i]],
                                    add=True)

                    @pl.when(i < GD // 2 - 1)
                    def _():
                        gstart(hp, b0 + 2, 0)

                    gwait(hp, b0 + 1, 1)
                    pltpu.sync_copy(ring.at[1], acc.at[didx[sl].at[2 * i + 1]],
                                    add=True)

        chunks = [(g, c) for g in (0, 1) for c in range(nc)]
        sched = (chunks[0::2], chunks[1::2])
        idx_pairs = ((srcg, dstg), (srcd, dstd))
        zb = ring.at[0]
        for p in range(nc):
            _fill(zb, C, 0.0)
            for off, n in _row_pieces(K):
                pltpu.async_copy(zb.at[pl.ds(0, n)],
                                 acc.at[pl.ds(r0 + off, n)], gsems[0])
            for off, n in _row_pieces(K):
                pltpu.make_async_copy(zb.at[pl.ds(0, n)],
                                      acc.at[pl.ds(r0 + off, n)],
                                      gsems[0]).wait()
            plsc.subcore_barrier()
            for core in (0, 1):
                g, c = sched[core][p]

                @pl.when(cid == core)
                def _(hp=hps[g * nc + c], sh=idx_pairs[g][0],
                      dh=idx_pairs[g][1]):
                    run_pass(hp, sh, dh)

            plsc.subcore_barrier()
            for core in (0, 1):
                g, c = sched[core][p]

                @pl.when(cid == core)
                def _(out=outs[g * nc + c]):
                    pieces = _row_pieces(K)
                    for k, (off, n) in enumerate(pieces):
                        sl = k % 2
                        if k >= 2:
                            poff, pn = pieces[k - 2]
                            pltpu.make_async_copy(
                                ring.at[sl].at[pl.ds(0, pn)],
                                out.at[pl.ds(r0 + poff, pn)],
                                isems[sl]).wait()
                        pltpu.async_copy(acc.at[pl.ds(r0 + off, n)],
                                         ring.at[sl].at[pl.ds(0, n)],
                                         gsems[sl])
                        pltpu.make_async_copy(acc.at[pl.ds(r0 + off, n)],
                                              ring.at[sl].at[pl.ds(0, n)],
                                              gsems[sl]).wait()
                        pltpu.async_copy(ring.at[sl].at[pl.ds(0, n)],
                                         out.at[pl.ds(r0 + off, n)],
                                         isems[sl])
                    for k in (len(pieces) - 2, len(pieces) - 1):
                        off, n = pieces[k]
                        pltpu.make_async_copy(ring.at[k % 2].at[pl.ds(0, n)],
                                              out.at[pl.ds(r0 + off, n)],
                                              isems[k % 2]).wait()

    return pl.kernel(body, out_type=out_type, mesh=_MESH,
                     scratch_types=scratch)


def _tcpre(r, Wg, Wd, cntg, cntd, apply_relu, cw):
    din = r.shape[1]
    dout = Wg.shape[1]
    nc = dout // cw

    def body(r_ref, wg_ref, wd_ref, cg_ref, cd_ref, *out_refs):
        rb = r_ref[...]
        if apply_relu:
            rb = jnp.maximum(rb, 0.0)
        disg = lax.rsqrt(1.0 + cg_ref[...])
        disd = lax.rsqrt(1.0 + cd_ref[...])
        hg = jnp.dot(rb, wg_ref[...], preferred_element_type=jnp.float32) * disg
        hd = jnp.dot(rb, wd_ref[...], preferred_element_type=jnp.float32) * disd
        for c in range(nc):
            out_refs[c][...] = hg[:, c * cw:(c + 1) * cw]
            out_refs[nc + c][...] = hd[:, c * cw:(c + 1) * cw]

    return pl.pallas_call(
        body,
        grid=(N // BN,),
        in_specs=[
            pl.BlockSpec((BN, din), lambda i: (i, 0)),
            pl.BlockSpec((din, dout), lambda i: (0, 0)),
            pl.BlockSpec((din, dout), lambda i: (0, 0)),
            pl.BlockSpec((BN, 1), lambda i: (i, 0)),
            pl.BlockSpec((BN, 1), lambda i: (i, 0)),
        ],
        out_specs=[pl.BlockSpec((BN, cw), lambda i: (i, 0))] * (2 * nc),
        out_shape=[jax.ShapeDtypeStruct((N, cw), jnp.float32)] * (2 * nc),
    )(r, Wg, Wd, cntg, cntd)


def _tcpost(accs, hps, cntg, cntd, bg, bd, cw):
    nc = len(accs) // 2
    dout = nc * cw

    def body(*refs):
        accr = refs[:2 * nc]
        hpr = refs[2 * nc:4 * nc]
        cg, cd, bgr, bdr, out = refs[4 * nc:]
        disg = lax.rsqrt(1.0 + cg[...])
        disd = lax.rsqrt(1.0 + cd[...])
        bias = bgr[...] + bdr[...]
        cols = []
        for c in range(nc):
            xc = (disg * (accr[c][...] + hpr[c][...])
                  + disd * (accr[nc + c][...] + hpr[nc + c][...])
                  + bias[:, c * cw:(c + 1) * cw])
            cols.append(xc)
        out[...] = cols[0] if nc == 1 else jnp.concatenate(cols, axis=1)

    in_specs = (
        [pl.BlockSpec((BN, cw), lambda i: (i, 0))] * (2 * nc)
        + [pl.BlockSpec((BN, cw), lambda i: (i, 0))] * (2 * nc)
        + [pl.BlockSpec((BN, 1), lambda i: (i, 0)),
           pl.BlockSpec((BN, 1), lambda i: (i, 0)),
           pl.BlockSpec((1, dout), lambda i: (0, 0)),
           pl.BlockSpec((1, dout), lambda i: (0, 0))]
    )
    return pl.pallas_call(
        body,
        grid=(N // BN,),
        in_specs=in_specs,
        out_specs=pl.BlockSpec((BN, dout), lambda i: (i, 0)),
        out_shape=jax.ShapeDtypeStruct((N, dout), jnp.float32),
    )(*accs, *hps, cntg, cntd, bg, bd)


def _pad_edges(adj):
    src = adj[0].astype(jnp.int32)
    dst = adj[1].astype(jnp.int32)
    pad = EPAD - src.shape[0]
    src = jnp.concatenate([src, jnp.zeros((pad,), jnp.int32)])
    dst = jnp.concatenate([dst, jnp.full((pad,), N, jnp.int32)])
    return src.reshape(NTILES, EPT), dst.reshape(NTILES, NB, K)


def kernel(x, adj_t_gg, adj_t_gd, emb,
           W1_gg, b1_gg, W2_gg, b2_gg, W3_gg, b3_gg,
           W1_gd, b1_gd, W2_gd, b2_gd, W3_gd, b3_gd):
    xe = jnp.concatenate([x, emb], axis=0)
    srcg, dstg = _pad_edges(adj_t_gg)
    srcd, dstd = _pad_edges(adj_t_gd)

    cntg16, cntd16 = _deg_kernel(dstg, dstd)
    cntg = lax.slice(cntg16, (0, 0), (N, 1))
    cntd = lax.slice(cntd16, (0, 0), (N, 1))

    layers = (
        (W1_gg, b1_gg, W1_gd, b1_gd),
        (W2_gg, b2_gg, W2_gd, b2_gd),
        (W3_gg, b3_gg, W3_gd, b3_gd),
    )
    r = xe
    outs = []
    for li, (Wg, bg, Wd, bd) in enumerate(layers):
        nc = Wg.shape[1] // C
        hps = _tcpre(r, Wg, Wd, cntg, cntd, apply_relu=(li > 0), cw=C)
        accs = _make_agg(nc)(*hps, srcg, dstg, srcd, dstd)
        xl = _tcpost(tuple(accs), tuple(hps), cntg, cntd,
                     bg.reshape(1, -1), bd.reshape(1, -1), cw=C)
        outs.append(xl)
        r = xl
    return jnp.concatenate(outs, axis=-1)
